# baseline (speedup 1.0000x reference)
import functools

import jax
import jax.numpy as jnp
from jax import lax
from jax.experimental import pallas as pl
from jax.experimental.pallas import tpu as pltpu
from jax.experimental.pallas import tpu_sc as plsc

_CB = 8192
_D = 32
_MBLK = 1024
_KBLK = 1024
_NKB = _CB // _KBLK


_SL = 8


def _dist_body(xt_ref, e_ref, ind_ref, loss_ref, s_ref, e2_ref):
    xt = xt_ref[...]
    x2 = jnp.sum(xt * xt, axis=0)[None, :]

    xbt2 = (2.0 * xt).astype(jnp.bfloat16)
    iota_s = lax.broadcasted_iota(jnp.int32, (_SL, _MBLK), 0)

    def chunk(c):
        e_c = e_ref[c * _KBLK:(c + 1) * _KBLK, :]
        s_ref[...] = jnp.dot(e_c.astype(jnp.bfloat16), xbt2,
                             preferred_element_type=jnp.float32)
        e2_ref[...] = jnp.sum(e_c * e_c, axis=1)[:, None]

        def scan(r, carry):
            run_v, run_r = carry
            s_r = s_ref[pl.ds(r * _SL, _SL), :]
            e2_r = e2_ref[pl.ds(r * _SL, _SL), :]
            q_r = (x2 - s_r) + e2_r
            m = q_r < run_v
            return (jnp.where(m, q_r, run_v), jnp.where(m, r, run_r))

        init = (jnp.full((_SL, _MBLK), jnp.inf, jnp.float32),
                jnp.zeros((_SL, _MBLK), jnp.int32))
        run_v, run_r = lax.fori_loop(0, _KBLK // _SL, scan, init)
        run_j = run_r * _SL + iota_s + c * _KBLK

        def comb(av, aj, bv, bj):
            take_a = (av < bv) | ((av == bv) & (aj < bj))
            return jnp.where(take_a, av, bv), jnp.where(take_a, aj, bj)

        v, j = run_v, run_j
        for h in (4, 2, 1):
            v, j = comb(v[:h], j[:h], v[h:2 * h], j[h:2 * h])
        return v[0], j[0]

    def half_argmin(lo, hi):
        run_min, run_idx = chunk(lo)
        for c in range(lo + 1, hi):
            cmin, cidx = chunk(c)
            upd = cmin < run_min
            run_idx = jnp.where(upd, cidx, run_idx)
            run_min = jnp.where(upd, cmin, run_min)
        return run_min, run_idx

    m1, i1 = half_argmin(0, _NKB // 2)
    m2, i2 = half_argmin(_NKB // 2, _NKB)
    m1b = m1.astype(jnp.bfloat16).astype(jnp.float32)
    keep1 = m1b <= m2
    run_idx = jnp.where(keep1, i1, i2)
    run_min = jnp.where(keep1, m1, m2)
    ind_ref[...] = run_idx

    sqd = run_min
    i = pl.program_id(0)
    nprog = pl.num_programs(0)
    prev = jnp.where(i == 0, 0.0, loss_ref[0, 0])
    tot = prev + jnp.sum(sqd)
    denom = jnp.float32(nprog * _MBLK * _D)
    loss_ref[0, 0] = jnp.where(i == nprog - 1, tot / denom, tot)


def _argmin_and_loss(flat_t, embed):
    m = flat_t.shape[1]
    return pl.pallas_call(
        _dist_body,
        grid=(m // _MBLK,),
        in_specs=[
            pl.BlockSpec((_D, _MBLK), lambda i: (0, i)),
            pl.BlockSpec((_CB, _D), lambda i: (0, 0)),
        ],
        out_specs=[
            pl.BlockSpec((_MBLK,), lambda i: (i,)),
            pl.BlockSpec(memory_space=pltpu.SMEM),
        ],
        out_shape=[
            jax.ShapeDtypeStruct((m,), jnp.int32),
            jax.ShapeDtypeStruct((1, 1), jnp.float32),
        ],
        scratch_shapes=[
            pltpu.VMEM((_KBLK, _MBLK), jnp.float32),
            pltpu.VMEM((_KBLK, 1), jnp.float32),
        ],
    )(flat_t, embed)


_IBLK = 128


@functools.cache
def _sc_gather(b_total):
    info = plsc.get_sparse_core_info()
    nc, ns = info.num_cores, info.num_subcores
    nw = nc * ns
    b_per_w = b_total // nw
    nj = b_per_w // _IBLK
    mesh = plsc.VectorSubcoreMesh(core_axis_name="c", subcore_axis_name="s")

    @functools.partial(
        pl.kernel, mesh=mesh,
        compiler_params=pltpu.CompilerParams(use_tc_tiling_on_sc=False),
        out_type=jax.ShapeDtypeStruct((b_total, _D), jnp.float32),
        scratch_types=[
            pltpu.VMEM((nj, _IBLK), jnp.int32),
            pltpu.VMEM((b_per_w, _D), jnp.float32),
            pltpu.SemaphoreType.DMA,
        ],
    )
    def gather(table_hbm, idx_hbm, out_hbm, idx_v, rows_v, sem):
        wid = lax.axis_index("s") * nc + lax.axis_index("c")
        pltpu.sync_copy(idx_hbm.at[pl.ds(wid * nj, nj)], idx_v)
        copies = [
            pltpu.async_copy(table_hbm.at[idx_v.at[j]],
                             rows_v.at[pl.ds(j * _IBLK, _IBLK)], sem)
            for j in range(nj)
        ]
        for c in copies:
            c.wait()
        pltpu.sync_copy(rows_v, out_hbm.at[pl.ds(wid * b_per_w, b_per_w)])

    return gather


def kernel(x, embed):
    b, n, d = x.shape
    flat = x.reshape(-1, d)
    ind, loss = _argmin_and_loss(flat.T, embed)
    quant = _sc_gather(flat.shape[0])(embed, ind.reshape(-1, _IBLK))
    return quant.reshape(b, n, d), ind.reshape(b, n), loss[0, 0]


# speedup vs baseline: 8.5168x; 8.5168x over previous
import functools

import jax
import jax.numpy as jnp
from jax import lax
from jax.experimental import pallas as pl
from jax.experimental.pallas import tpu as pltpu
from jax.experimental.pallas import tpu_sc as plsc

_CB = 8192
_D = 32
_MBLK = 1024
_KBLK = 1024
_NKB = _CB // _KBLK


_SL = 8


def _dist_body(xt_ref, e_ref, ind_ref, loss_ref, s_ref, e2_ref):
    xt = xt_ref[...]
    x2 = jnp.sum(xt * xt, axis=0)[None, :]

    xbt2 = (2.0 * xt).astype(jnp.bfloat16)
    iota_s = lax.broadcasted_iota(jnp.int32, (_SL, _MBLK), 0)

    def chunk(c):
        e_c = e_ref[c * _KBLK:(c + 1) * _KBLK, :]
        s_ref[...] = jnp.dot(e_c.astype(jnp.bfloat16), xbt2,
                             preferred_element_type=jnp.float32)
        e2_ref[...] = jnp.sum(e_c * e_c, axis=1)[:, None]

        run_v = jnp.full((_SL, _MBLK), jnp.inf, jnp.float32)
        run_r = jnp.zeros((_SL, _MBLK), jnp.int32)
        for r in range(_KBLK // _SL):
            s_r = s_ref[r * _SL:(r + 1) * _SL, :]
            e2_r = e2_ref[r * _SL:(r + 1) * _SL, :]
            q_r = (x2 - s_r) + e2_r
            m = q_r < run_v
            run_v = jnp.where(m, q_r, run_v)
            run_r = jnp.where(m, r, run_r)
        run_j = run_r * _SL + iota_s + c * _KBLK

        def comb(av, aj, bv, bj):
            take_a = (av < bv) | ((av == bv) & (aj < bj))
            return jnp.where(take_a, av, bv), jnp.where(take_a, aj, bj)

        v, j = run_v, run_j
        for h in (4, 2, 1):
            v, j = comb(v[:h], j[:h], v[h:2 * h], j[h:2 * h])
        return v[0], j[0]

    def half_argmin(lo, hi):
        run_min, run_idx = chunk(lo)
        for c in range(lo + 1, hi):
            cmin, cidx = chunk(c)
            upd = cmin < run_min
            run_idx = jnp.where(upd, cidx, run_idx)
            run_min = jnp.where(upd, cmin, run_min)
        return run_min, run_idx

    m1, i1 = half_argmin(0, _NKB // 2)
    m2, i2 = half_argmin(_NKB // 2, _NKB)
    m1b = m1.astype(jnp.bfloat16).astype(jnp.float32)
    keep1 = m1b <= m2
    run_idx = jnp.where(keep1, i1, i2)
    run_min = jnp.where(keep1, m1, m2)
    ind_ref[...] = run_idx

    sqd = run_min
    i = pl.program_id(0)
    nprog = pl.num_programs(0)
    prev = jnp.where(i == 0, 0.0, loss_ref[0, 0])
    tot = prev + jnp.sum(sqd)
    denom = jnp.float32(nprog * _MBLK * _D)
    loss_ref[0, 0] = jnp.where(i == nprog - 1, tot / denom, tot)


def _argmin_and_loss(flat_t, embed):
    m = flat_t.shape[1]
    return pl.pallas_call(
        _dist_body,
        grid=(m // _MBLK,),
        in_specs=[
            pl.BlockSpec((_D, _MBLK), lambda i: (0, i)),
            pl.BlockSpec((_CB, _D), lambda i: (0, 0)),
        ],
        out_specs=[
            pl.BlockSpec((_MBLK,), lambda i: (i,)),
            pl.BlockSpec(memory_space=pltpu.SMEM),
        ],
        out_shape=[
            jax.ShapeDtypeStruct((m,), jnp.int32),
            jax.ShapeDtypeStruct((1, 1), jnp.float32),
        ],
        scratch_shapes=[
            pltpu.VMEM((_KBLK, _MBLK), jnp.float32),
            pltpu.VMEM((_KBLK, 1), jnp.float32),
        ],
    )(flat_t, embed)


_IBLK = 128


@functools.cache
def _sc_gather(b_total):
    info = plsc.get_sparse_core_info()
    nc, ns = info.num_cores, info.num_subcores
    nw = nc * ns
    b_per_w = b_total // nw
    nj = b_per_w // _IBLK
    mesh = plsc.VectorSubcoreMesh(core_axis_name="c", subcore_axis_name="s")

    @functools.partial(
        pl.kernel, mesh=mesh,
        compiler_params=pltpu.CompilerParams(use_tc_tiling_on_sc=False),
        out_type=jax.ShapeDtypeStruct((b_total, _D), jnp.float32),
        scratch_types=[
            pltpu.VMEM((nj, _IBLK), jnp.int32),
            pltpu.VMEM((b_per_w, _D), jnp.float32),
            pltpu.SemaphoreType.DMA,
        ],
    )
    def gather(table_hbm, idx_hbm, out_hbm, idx_v, rows_v, sem):
        wid = lax.axis_index("s") * nc + lax.axis_index("c")
        pltpu.sync_copy(idx_hbm.at[pl.ds(wid * nj, nj)], idx_v)
        copies = [
            pltpu.async_copy(table_hbm.at[idx_v.at[j]],
                             rows_v.at[pl.ds(j * _IBLK, _IBLK)], sem)
            for j in range(nj)
        ]
        for c in copies:
            c.wait()
        pltpu.sync_copy(rows_v, out_hbm.at[pl.ds(wid * b_per_w, b_per_w)])

    return gather


def kernel(x, embed):
    b, n, d = x.shape
    flat = x.reshape(-1, d)
    ind, loss = _argmin_and_loss(flat.T, embed)
    quant = _sc_gather(flat.shape[0])(embed, ind.reshape(-1, _IBLK))
    return quant.reshape(b, n, d), ind.reshape(b, n), loss[0, 0]
